# Initial kernel scaffold; baseline (speedup 1.0000x reference)
import jax
import jax.numpy as jnp
from jax.experimental import pallas as pl


def kernel(probas, labels):
    raise NotImplementedError("write your pallas kernel here")


# speedup vs baseline: 33.1917x; 33.1917x over previous
import functools

import jax
import jax.numpy as jnp
from jax import lax
from jax.experimental import pallas as pl
from jax.experimental.pallas import tpu as pltpu
from jax.experimental.pallas import tpu_sc as plsc

NB = 2048
MBITS = 6
SHIFT = 23 - MBITS
BASE = (127 - 16) << MBITS
NCLASS = 21
LANES = 16


def _sc_hist_kernel(probas_hbm, labels_hbm, out_hbm, lab_v, prob_v, hist_v):
    nc = 2
    cid = lax.axis_index("c")
    sid = lax.axis_index("s")
    wid = sid * nc + cid
    chunk = lab_v.shape[0]
    plane = 8 * chunk
    b = wid // 8
    off = (wid % 8) * chunk

    pltpu.sync_copy(labels_hbm.at[pl.ds(b * plane + off, chunk)], lab_v)

    ones = jnp.ones((LANES,), jnp.float32)
    zeros = jnp.zeros((LANES,), jnp.float32)

    for c in range(NCLASS):
        pltpu.sync_copy(
            probas_hbm.at[pl.ds((b * NCLASS + c) * plane + off, chunk)], prob_v)

        def zero_body(i, carry):
            hist_v[pl.ds(i * LANES, LANES)] = zeros
            return carry

        lax.fori_loop(0, (4 * NB) // LANES, zero_body, 0)

        def body(i, carry):
            base = i * LANES
            x = prob_v[pl.ds(base, LANES)]
            l = lab_v[pl.ds(base, LANES)]
            p = jnp.exp(x)
            fg = l == c
            e = jnp.where(fg, jnp.abs(1.0 - p), p)
            bits = lax.bitcast_convert_type(e, jnp.int32)
            bk = lax.shift_right_logical(bits, SHIFT) - BASE
            bk = jnp.clip(bk, 0, NB - 1)
            cnt_idx = bk + jnp.where(fg, 0, 2 * NB)
            plsc.addupdate_scatter(hist_v, [cnt_idx], ones)
            plsc.addupdate_scatter(hist_v, [cnt_idx + NB], e)
            return carry

        lax.fori_loop(0, chunk // LANES, body, 0)
        pltpu.sync_copy(hist_v, out_hbm.at[pl.ds((wid * NCLASS + c) * 4 * NB, 4 * NB)])


def _suffix_excl(x):
    c, nb = x.shape
    s = x
    k = 1
    while k < nb:
        pad = jnp.zeros((c, k), x.dtype)
        s = s + jnp.concatenate([s[:, k:], pad], axis=1)
        k *= 2
    return s - x


def _tc_finalize_kernel(hist_ref, out_ref):
    h = hist_ref[...]
    hs = jnp.sum(h, axis=0)
    cnt_fg = hs[:, 0, :]
    esum_fg = hs[:, 1, :]
    cnt_bg = hs[:, 2, :]
    esum_bg = hs[:, 3, :]
    sb = _suffix_excl(cnt_bg)
    sf = _suffix_excl(cnt_fg)
    p = jnp.sum(cnt_fg, axis=1, keepdims=True)
    d1 = jnp.maximum(p + sb + 0.5 * cnt_bg, 0.5)
    term_fg = esum_fg / d1
    d2 = jnp.maximum((p + sb) * (p + sb + cnt_bg), 0.5)
    term_bg = esum_bg * (p - sf - 0.5 * cnt_fg) / d2
    losses = jnp.sum(term_fg + term_bg, axis=1)
    present = (p[:, 0] > 0).astype(jnp.float32)
    denom = jnp.maximum(jnp.sum(present), 1.0)
    out_ref[...] = (jnp.sum(losses * present) / denom).reshape(1, 1)


def kernel(probas, labels):
    B, C, H, W = probas.shape
    chunk = (B * H * W) // 32
    probas1 = probas.reshape(-1)
    labels1 = labels.reshape(-1)

    mesh = plsc.VectorSubcoreMesh(core_axis_name="c", subcore_axis_name="s")
    hist = pl.kernel(
        _sc_hist_kernel,
        mesh=mesh,
        compiler_params=pltpu.CompilerParams(needs_layout_passes=False),
        out_type=jax.ShapeDtypeStruct((32 * NCLASS * 4 * NB,), jnp.float32),
        scratch_types=[
            pltpu.VMEM((chunk,), jnp.int32),
            pltpu.VMEM((chunk,), jnp.float32),
            pltpu.VMEM((4 * NB,), jnp.float32),
        ],
    )(probas1, labels1)

    hist4 = hist.reshape(32, NCLASS, 4, NB)
    out = pl.pallas_call(
        _tc_finalize_kernel,
        out_shape=jax.ShapeDtypeStruct((1, 1), jnp.float32),
    )(hist4)
    return out.reshape(())


# speedup vs baseline: 114.3548x; 3.4453x over previous
import functools

import jax
import jax.numpy as jnp
from jax import lax
from jax.experimental import pallas as pl
from jax.experimental.pallas import tpu as pltpu
from jax.experimental.pallas import tpu_sc as plsc

NB = 2048
MBITS = 6
SHIFT = 23 - MBITS
BASE = (127 - 16) << MBITS
NCLASS = 21
LANES = 16


def _sc_hist_kernel(probas_hbm, labels_hbm, out_hbm, lab_v,
                    prob_a, prob_b, hist_a, hist_b,
                    isem_a, isem_b, osem_a, osem_b):
    nc = 2
    cid = lax.axis_index("c")
    sid = lax.axis_index("s")
    wid = sid * nc + cid
    chunk = lab_v.shape[0]
    plane = 8 * chunk
    b = wid // 8
    off = (wid % 8) * chunk

    pltpu.sync_copy(labels_hbm.at[pl.ds(b * plane + off, chunk)], lab_v)

    ones = jnp.ones((LANES,), jnp.float32)
    zeros = jnp.zeros((LANES,), jnp.float32)
    prob_bufs = (prob_a, prob_b)
    hist_bufs = (hist_a, hist_b)
    isems = (isem_a, isem_b)
    osems = (osem_a, osem_b)

    def start_in(c):
        return pltpu.async_copy(
            probas_hbm.at[pl.ds((b * NCLASS + c) * plane + off, chunk)],
            prob_bufs[c % 2], isems[c % 2])

    in_handles = {0: start_in(0)}
    out_handles = {}
    for c in range(NCLASS):
        if c + 1 < NCLASS:
            in_handles[c + 1] = start_in(c + 1)
        prob_v = prob_bufs[c % 2]
        hist_v = hist_bufs[c % 2]
        if c >= 2:
            out_handles[c - 2].wait()

        @plsc.parallel_loop(0, (4 * NB) // LANES, unroll=8)
        def zero_body(i, hist_v=hist_v):
            hist_v[pl.ds(i * LANES, LANES)] = zeros

        in_handles[c].wait()

        @plsc.parallel_loop(0, chunk // LANES, unroll=8)
        def body(i, prob_v=prob_v, hist_v=hist_v, c=c):
            base = i * LANES
            x = prob_v[pl.ds(base, LANES)]
            l = lab_v[pl.ds(base, LANES)]
            p = jnp.exp(x)
            fg = l == c
            e = jnp.where(fg, jnp.abs(1.0 - p), p)
            bits = lax.bitcast_convert_type(e, jnp.int32)
            bk = lax.shift_right_logical(bits, SHIFT) - BASE
            bk = jnp.clip(bk, 0, NB - 1)
            cnt_idx = bk + jnp.where(fg, 0, 2 * NB)
            plsc.addupdate_scatter(hist_v, [cnt_idx], ones)
            plsc.addupdate_scatter(hist_v, [cnt_idx + NB], e)

        out_handles[c] = pltpu.async_copy(
            hist_v, out_hbm.at[pl.ds((wid * NCLASS + c) * 4 * NB, 4 * NB)],
            osems[c % 2])
    out_handles[NCLASS - 2].wait()
    out_handles[NCLASS - 1].wait()


def _suffix_excl(x):
    c, nb = x.shape
    s = x
    k = 1
    while k < nb:
        pad = jnp.zeros((c, k), x.dtype)
        s = s + jnp.concatenate([s[:, k:], pad], axis=1)
        k *= 2
    return s - x


def _tc_finalize_kernel(hist_ref, out_ref):
    h = hist_ref[...]
    hs = jnp.sum(h, axis=0)
    cnt_fg = hs[:, 0, :]
    esum_fg = hs[:, 1, :]
    cnt_bg = hs[:, 2, :]
    esum_bg = hs[:, 3, :]
    sb = _suffix_excl(cnt_bg)
    sf = _suffix_excl(cnt_fg)
    p = jnp.sum(cnt_fg, axis=1, keepdims=True)
    d1 = jnp.maximum(p + sb + 0.5 * cnt_bg, 0.5)
    term_fg = esum_fg / d1
    d2 = jnp.maximum((p + sb) * (p + sb + cnt_bg), 0.5)
    term_bg = esum_bg * (p - sf - 0.5 * cnt_fg) / d2
    losses = jnp.sum(term_fg + term_bg, axis=1)
    present = (p[:, 0] > 0).astype(jnp.float32)
    denom = jnp.maximum(jnp.sum(present), 1.0)
    out_ref[...] = (jnp.sum(losses * present) / denom).reshape(1, 1)


def kernel(probas, labels):
    B, C, H, W = probas.shape
    chunk = (B * H * W) // 32
    probas1 = probas.reshape(-1)
    labels1 = labels.reshape(-1)

    mesh = plsc.VectorSubcoreMesh(core_axis_name="c", subcore_axis_name="s")
    hist = pl.kernel(
        _sc_hist_kernel,
        mesh=mesh,
        compiler_params=pltpu.CompilerParams(needs_layout_passes=False),
        out_type=jax.ShapeDtypeStruct((32 * NCLASS * 4 * NB,), jnp.float32),
        scratch_types=[
            pltpu.VMEM((chunk,), jnp.int32),
            pltpu.VMEM((chunk,), jnp.float32),
            pltpu.VMEM((chunk,), jnp.float32),
            pltpu.VMEM((4 * NB,), jnp.float32),
            pltpu.VMEM((4 * NB,), jnp.float32),
            pltpu.SemaphoreType.DMA,
            pltpu.SemaphoreType.DMA,
            pltpu.SemaphoreType.DMA,
            pltpu.SemaphoreType.DMA,
        ],
    )(probas1, labels1)

    hist4 = hist.reshape(32, NCLASS, 4, NB)
    out = pl.pallas_call(
        _tc_finalize_kernel,
        out_shape=jax.ShapeDtypeStruct((1, 1), jnp.float32),
    )(hist4)
    return out.reshape(())
